# baseline (speedup 1.0000x reference)
import functools

import jax
import jax.numpy as jnp
from jax import lax
from jax.experimental import pallas as pl
from jax.experimental.pallas import tpu as pltpu
from jax.experimental.pallas import tpu_sc as plsc

_N_FIELDS = 26
_VOCAB = 100000
_EMBED_DIM = 16
_BATCH = 16384
_HASH_BUCKETS = 1000

_NW = 32
_BPW = _BATCH // _NW


def _hash_body(idx_ref, h_ref):
    xu = idx_ref[...].astype(jnp.uint32)
    h = (xu * jnp.uint32(2654435761)) % jnp.uint32(_HASH_BUCKETS)
    h_ref[...] = h.astype(jnp.int32)


def _hash_call(indices):
    return pl.pallas_call(
        _hash_body,
        out_shape=jax.ShapeDtypeStruct((_BATCH, _N_FIELDS), jnp.int32),
    )(indices)


_MESH = plsc.VectorSubcoreMesh(core_axis_name="c", subcore_axis_name="s")


@functools.partial(
    pl.kernel,
    mesh=_MESH,
    compiler_params=pltpu.CompilerParams(use_tc_tiling_on_sc=False),
    out_type=jax.ShapeDtypeStruct((_BATCH, _N_FIELDS * _EMBED_DIM), jnp.float32),
    scratch_types=[
        pltpu.VMEM((2, _BPW), jnp.int32),
        pltpu.VMEM((2, _BPW, _EMBED_DIM), jnp.float32),
        pltpu.SemaphoreType.DMA,
        pltpu.SemaphoreType.DMA,
        pltpu.SemaphoreType.DMA,
        pltpu.SemaphoreType.DMA,
    ],
)
def _sc_gather(idxt_hbm, tab_hbm, out_hbm, idx_v, rows_v, g0, g1, s0, s1):
    gsem = (g0, g1)
    ssem = (s0, s1)
    wid = lax.axis_index("s") * 2 + lax.axis_index("c")
    b0 = wid * _BPW

    gathers = [None] * _N_FIELDS
    stores = [None] * _N_FIELDS
    for f in range(_N_FIELDS):
        b = f % 2
        if f >= 2:
            stores[f - 2].wait()
        pltpu.sync_copy(idxt_hbm.at[f, pl.ds(b0, _BPW)], idx_v.at[b])
        gathers[f] = pltpu.async_copy(
            tab_hbm.at[f].at[idx_v.at[b]], rows_v.at[b], gsem[b])
        if f >= 1:
            gathers[f - 1].wait()
            stores[f - 1] = pltpu.async_copy(
                rows_v.at[1 - b],
                out_hbm.at[pl.ds(b0, _BPW), pl.ds((f - 1) * _EMBED_DIM, _EMBED_DIM)],
                ssem[1 - b])
    last = _N_FIELDS - 1
    gathers[last].wait()
    stores[last] = pltpu.async_copy(
        rows_v.at[last % 2],
        out_hbm.at[pl.ds(b0, _BPW), pl.ds(last * _EMBED_DIM, _EMBED_DIM)],
        ssem[last % 2])
    stores[last - 1].wait()
    stores[last].wait()


def kernel(indices, tables):
    h = _hash_call(indices)
    idxt = indices.T
    out = _sc_gather(idxt, tables)
    return out, h
